# initial kernel scaffold (unmeasured)
import jax
import jax.numpy as jnp
from jax import lax
from jax.experimental import pallas as pl
from jax.experimental.pallas import tpu as pltpu

N_DEV = 8
HQ_LOC = 8
DH = 128
SQ = 1024
SKV = 1024
D_MODEL = 1024
SCALE = 0.08838834764831843


def kernel(x, Wq, K_ext, V_ext, Wo):
    pos = lax.axis_index("i")
    x2 = x[0]
    k_loc = lax.dynamic_slice_in_dim(K_ext[0], pos * HQ_LOC, HQ_LOC, axis=1)
    v_loc = lax.dynamic_slice_in_dim(V_ext[0], pos * HQ_LOC, HQ_LOC, axis=1)

    def body(x_ref, wq_ref, k_ref, v_ref, wo_ref, out_ref,
             ctx_ref, comm_ref, send_sems, recv_sems):
        my = lax.axis_index("i")
        left = lax.rem(my + N_DEV - 1, N_DEV)
        right = lax.rem(my + 1, N_DEV)

        barrier_sem = pltpu.get_barrier_semaphore()
        for nbr in (left, right):
            pl.semaphore_signal(
                barrier_sem, inc=1,
                device_id=(nbr,), device_id_type=pl.DeviceIdType.MESH,
            )
        pl.semaphore_wait(barrier_sem, 2)

        q = jnp.dot(x_ref[...], wq_ref[...], preferred_element_type=jnp.float32)

        qi = lax.broadcasted_iota(jnp.int32, (SQ, SKV), 0)
        ki = lax.broadcasted_iota(jnp.int32, (SQ, SKV), 1)
        mask = (jnp.abs(qi - ki) <= 128) | (ki < 32) | (qi < 32)
        bias = jnp.where(mask, 0.0, -1e9).astype(jnp.float32)

        for h in range(HQ_LOC):
            q_h = q[:, h * DH:(h + 1) * DH]
            k_h = k_ref[:, h, :]
            v_h = v_ref[:, h, :]
            s = lax.dot_general(
                q_h, k_h, (((1,), (1,)), ((), ())),
                preferred_element_type=jnp.float32,
            ) * SCALE + bias
            m = jnp.max(s, axis=-1, keepdims=True)
            w = jnp.exp(s - m)
            w = w / jnp.sum(w, axis=-1, keepdims=True)
            ctx_ref[:, h * DH:(h + 1) * DH] = jnp.dot(
                w, v_h, preferred_element_type=jnp.float32)

        partial = jnp.dot(ctx_ref[...], wo_ref[...],
                          preferred_element_type=jnp.float32)
        out_ref[...] = partial
        comm_ref[N_DEV - 1, :, :] = partial

        for h in range(N_DEV - 1):
            src_slot = h - 1 if h > 0 else N_DEV - 1
            rdma = pltpu.make_async_remote_copy(
                src_ref=comm_ref.at[src_slot],
                dst_ref=comm_ref.at[h],
                send_sem=send_sems.at[h],
                recv_sem=recv_sems.at[h],
                device_id=(right,),
                device_id_type=pl.DeviceIdType.MESH,
            )
            rdma.start()
            rdma.wait()
            out_ref[...] = out_ref[...] + comm_ref[h, :, :]

    out = pl.pallas_call(
        body,
        out_shape=jax.ShapeDtypeStruct((SQ, D_MODEL), jnp.float32),
        in_specs=[pl.BlockSpec(memory_space=pltpu.VMEM)] * 5,
        out_specs=pl.BlockSpec(memory_space=pltpu.VMEM),
        scratch_shapes=[
            pltpu.VMEM((SQ, HQ_LOC * DH), jnp.float32),
            pltpu.VMEM((N_DEV, SQ, D_MODEL), jnp.float32),
            pltpu.SemaphoreType.DMA((N_DEV - 1,)),
            pltpu.SemaphoreType.DMA((N_DEV - 1,)),
        ],
        compiler_params=pltpu.CompilerParams(collective_id=0),
    )(x2, Wq, k_loc, v_loc, Wo)
    return out[None]


# baseline (device time: 153385 ns/iter reference)
import jax
import jax.numpy as jnp
from jax import lax
from jax.experimental import pallas as pl
from jax.experimental.pallas import tpu as pltpu

N_DEV = 8
HQ_LOC = 8
DH = 128
SQ = 1024
SKV = 1024
D_MODEL = 1024
SCALE = 0.08838834764831843


def kernel(x, Wq, K_ext, V_ext, Wo):
    pos = lax.axis_index("i")
    x2 = x[0]
    k_loc = lax.dynamic_slice_in_dim(K_ext[0], pos * HQ_LOC, HQ_LOC, axis=1)
    v_loc = lax.dynamic_slice_in_dim(V_ext[0], pos * HQ_LOC, HQ_LOC, axis=1)

    def body(x_ref, wq_ref, k_ref, v_ref, wo_ref, out_ref,
             ctx_ref, stage_ref, rs_send, rs_recv, ag_send, ag_recv):
        my = lax.axis_index("i")
        left = lax.rem(my + N_DEV - 1, N_DEV)
        right = lax.rem(my + 1, N_DEV)

        barrier_sem = pltpu.get_barrier_semaphore()
        for nbr in (left, right):
            pl.semaphore_signal(
                barrier_sem, inc=1,
                device_id=(nbr,), device_id_type=pl.DeviceIdType.MESH,
            )
        pl.semaphore_wait(barrier_sem, 2)

        q = jnp.dot(x_ref[...], wq_ref[...], preferred_element_type=jnp.float32)

        qi = lax.broadcasted_iota(jnp.int32, (SQ, SKV), 0)
        ki = lax.broadcasted_iota(jnp.int32, (SQ, SKV), 1)
        mask = (jnp.abs(qi - ki) <= 128) | (ki < 32) | (qi < 32)
        bias = jnp.where(mask, 0.0, -1e9).astype(jnp.float32)

        for h in range(HQ_LOC):
            q_h = q[:, h * DH:(h + 1) * DH]
            k_h = k_ref[:, h, :]
            v_h = v_ref[:, h, :]
            s = lax.dot_general(
                q_h, k_h, (((1,), (1,)), ((), ())),
                preferred_element_type=jnp.float32,
            ) * SCALE + bias
            m = jnp.max(s, axis=-1, keepdims=True)
            w = jnp.exp(s - m)
            w = w / jnp.sum(w, axis=-1, keepdims=True)
            ctx_ref[:, h * DH:(h + 1) * DH] = jnp.dot(
                w, v_h, preferred_element_type=jnp.float32)

        out_ref[...] = jnp.dot(ctx_ref[...], wo_ref[...],
                               preferred_element_type=jnp.float32)

        CH = SQ // N_DEV
        for s in range(N_DEV - 1):
            send_c = lax.rem(my - s + N_DEV, N_DEV)
            recv_c = lax.rem(my - 1 - s + 2 * N_DEV, N_DEV)
            rdma = pltpu.make_async_remote_copy(
                src_ref=out_ref.at[pl.ds(send_c * CH, CH), :],
                dst_ref=stage_ref.at[s],
                send_sem=rs_send.at[s],
                recv_sem=rs_recv.at[s],
                device_id=(right,),
                device_id_type=pl.DeviceIdType.MESH,
            )
            rdma.start()
            rdma.wait()
            out_ref[pl.ds(recv_c * CH, CH), :] = (
                out_ref[pl.ds(recv_c * CH, CH), :] + stage_ref[s, :, :])

        for s in range(N_DEV - 1):
            send_c = lax.rem(my + 1 - s + N_DEV, N_DEV)
            rdma = pltpu.make_async_remote_copy(
                src_ref=out_ref.at[pl.ds(send_c * CH, CH), :],
                dst_ref=out_ref.at[pl.ds(send_c * CH, CH), :],
                send_sem=ag_send.at[s],
                recv_sem=ag_recv.at[s],
                device_id=(right,),
                device_id_type=pl.DeviceIdType.MESH,
            )
            rdma.start()
            rdma.wait()

    out = pl.pallas_call(
        body,
        out_shape=jax.ShapeDtypeStruct((SQ, D_MODEL), jnp.float32),
        in_specs=[pl.BlockSpec(memory_space=pltpu.VMEM)] * 5,
        out_specs=pl.BlockSpec(memory_space=pltpu.VMEM),
        scratch_shapes=[
            pltpu.VMEM((SQ, HQ_LOC * DH), jnp.float32),
            pltpu.VMEM((N_DEV - 1, SQ // N_DEV, D_MODEL), jnp.float32),
            pltpu.SemaphoreType.DMA((N_DEV - 1,)),
            pltpu.SemaphoreType.DMA((N_DEV - 1,)),
            pltpu.SemaphoreType.DMA((N_DEV - 1,)),
            pltpu.SemaphoreType.DMA((N_DEV - 1,)),
        ],
        compiler_params=pltpu.CompilerParams(
            collective_id=0, vmem_limit_bytes=96 * 1024 * 1024),
    )(x2, Wq, k_loc, v_loc, Wo)
    return out[None]


# device time: 113629 ns/iter; 1.3499x vs baseline; 1.3499x over previous
import jax
import jax.numpy as jnp
from jax import lax
from jax.experimental import pallas as pl
from jax.experimental.pallas import tpu as pltpu

N_DEV = 8
HQ_LOC = 8
DH = 128
SQ = 1024
SKV = 1024
D_MODEL = 1024
SCALE = 0.08838834764831843


def kernel(x, Wq, K_ext, V_ext, Wo):
    pos = lax.axis_index("i")
    x2 = x[0]
    k_loc = lax.dynamic_slice_in_dim(K_ext[0], pos * HQ_LOC, HQ_LOC, axis=1)
    v_loc = lax.dynamic_slice_in_dim(V_ext[0], pos * HQ_LOC, HQ_LOC, axis=1)

    def body(x_ref, wq_ref, k_ref, v_ref, wo_ref, out_ref,
             ctx_ref, stage_ref, p1_send, p1_recv, p2_send, p2_recv):
        my = lax.axis_index("i")

        barrier_sem = pltpu.get_barrier_semaphore()
        for t in range(1, N_DEV):
            pl.semaphore_signal(
                barrier_sem, inc=1,
                device_id=(lax.rem(my + t, N_DEV),),
                device_id_type=pl.DeviceIdType.MESH,
            )
        pl.semaphore_wait(barrier_sem, N_DEV - 1)

        q = jnp.dot(x_ref[...], wq_ref[...], preferred_element_type=jnp.float32)

        qi = lax.broadcasted_iota(jnp.int32, (SQ, SKV), 0)
        ki = lax.broadcasted_iota(jnp.int32, (SQ, SKV), 1)
        mask = (jnp.abs(qi - ki) <= 128) | (ki < 32) | (qi < 32)
        bias = jnp.where(mask, 0.0, -1e9).astype(jnp.float32)

        for h in range(HQ_LOC):
            q_h = q[:, h * DH:(h + 1) * DH]
            k_h = k_ref[:, h, :]
            v_h = v_ref[:, h, :]
            s = lax.dot_general(
                q_h, k_h, (((1,), (1,)), ((), ())),
                preferred_element_type=jnp.float32,
            ) * SCALE + bias
            m = jnp.max(s, axis=-1, keepdims=True)
            w = jnp.exp(s - m)
            w = w / jnp.sum(w, axis=-1, keepdims=True)
            ctx_ref[:, h * DH:(h + 1) * DH] = jnp.dot(
                w, v_h, preferred_element_type=jnp.float32)

        CH = SQ // N_DEV
        sends = []
        for t in range(1, N_DEV):
            tgt = lax.rem(my + t, N_DEV)
            rows = pl.ds(tgt * CH, CH)
            out_ref[rows, :] = jnp.dot(
                ctx_ref[rows, :], wo_ref[...],
                preferred_element_type=jnp.float32)
            rdma = pltpu.make_async_remote_copy(
                src_ref=out_ref.at[rows, :],
                dst_ref=stage_ref.at[t],
                send_sem=p1_send.at[t],
                recv_sem=p1_recv.at[t],
                device_id=(tgt,),
                device_id_type=pl.DeviceIdType.MESH,
            )
            rdma.start()
            sends.append(rdma)

        my_rows = pl.ds(my * CH, CH)
        total = jnp.dot(ctx_ref[my_rows, :], wo_ref[...],
                        preferred_element_type=jnp.float32)
        for t in range(1, N_DEV):
            src = lax.rem(my - t + N_DEV, N_DEV)
            recv = pltpu.make_async_remote_copy(
                src_ref=out_ref.at[pl.ds(0, CH), :],
                dst_ref=stage_ref.at[t],
                send_sem=p1_send.at[t],
                recv_sem=p1_recv.at[t],
                device_id=(src,),
                device_id_type=pl.DeviceIdType.MESH,
            )
            recv.wait_recv()
            total = total + stage_ref[t, :, :]
        out_ref[my_rows, :] = total

        for t in range(1, N_DEV):
            tgt = lax.rem(my + t, N_DEV)
            rdma = pltpu.make_async_remote_copy(
                src_ref=out_ref.at[my_rows, :],
                dst_ref=out_ref.at[my_rows, :],
                send_sem=p2_send.at[t],
                recv_sem=p2_recv.at[t],
                device_id=(tgt,),
                device_id_type=pl.DeviceIdType.MESH,
            )
            rdma.start()
            sends.append(rdma)

        for t in range(1, N_DEV):
            src = lax.rem(my - t + N_DEV, N_DEV)
            recv = pltpu.make_async_remote_copy(
                src_ref=out_ref.at[pl.ds(0, CH), :],
                dst_ref=out_ref.at[pl.ds(src * CH, CH), :],
                send_sem=p2_send.at[t],
                recv_sem=p2_recv.at[t],
                device_id=(src,),
                device_id_type=pl.DeviceIdType.MESH,
            )
            recv.wait_recv()

        for rdma in sends:
            rdma.wait_send()

    out = pl.pallas_call(
        body,
        out_shape=jax.ShapeDtypeStruct((SQ, D_MODEL), jnp.float32),
        in_specs=[pl.BlockSpec(memory_space=pltpu.VMEM)] * 5,
        out_specs=pl.BlockSpec(memory_space=pltpu.VMEM),
        scratch_shapes=[
            pltpu.VMEM((SQ, HQ_LOC * DH), jnp.float32),
            pltpu.VMEM((N_DEV, SQ // N_DEV, D_MODEL), jnp.float32),
            pltpu.SemaphoreType.DMA((N_DEV,)),
            pltpu.SemaphoreType.DMA((N_DEV,)),
            pltpu.SemaphoreType.DMA((N_DEV,)),
            pltpu.SemaphoreType.DMA((N_DEV,)),
        ],
        compiler_params=pltpu.CompilerParams(
            collective_id=0, vmem_limit_bytes=96 * 1024 * 1024),
    )(x2, Wq, k_loc, v_loc, Wo)
    return out[None]
